# baseline (speedup 1.0000x reference)
import functools

import jax
import jax.numpy as jnp
from jax.experimental import pallas as pl

_TEMP = 0.05
_NB = 2048


def _logits_body(x_ref, f_ref, o_ref):
    x = x_ref[...]
    norm = jnp.sqrt(jnp.sum(x * x, axis=1, keepdims=True))
    xs = x * ((1.0 / _TEMP) / jnp.maximum(norm, 1e-12))
    o_ref[...] = jax.lax.dot_general(
        xs,
        f_ref[...],
        (((1,), (1,)), ((), ())),
        preferred_element_type=jnp.float32,
    )


@functools.partial(jax.jit, static_argnames=())
def kernel(inputs, targets, features):
    del targets
    b, d = inputs.shape
    n = features.shape[0]
    grid = (pl.cdiv(n, _NB),)
    return pl.pallas_call(
        _logits_body,
        grid=grid,
        in_specs=[
            pl.BlockSpec((b, d), lambda i: (0, 0)),
            pl.BlockSpec((_NB, d), lambda i: (i, 0)),
        ],
        out_specs=pl.BlockSpec((b, _NB), lambda i: (0, i)),
        out_shape=jax.ShapeDtypeStruct((b, n), jnp.float32),
    )(inputs, features)
